# baseline (speedup 1.0000x reference)
import functools

import jax
import jax.numpy as jnp
import numpy as np
from jax import lax
from jax.experimental import pallas as pl
from jax.experimental.pallas import tpu as pltpu
from jax.experimental.pallas import tpu_sc as plsc

N = 10000
NP = 10240
D = 128
E0P = 32768
E1P = 327680
NC0 = E0P // (32 * 128)
NC1 = E1P // (32 * 128)
ROWS_PER_TILE = NP // 16
INV_SQRT2 = np.float32(1.0 / np.sqrt(2.0))
F32 = jnp.float32
BF16 = jnp.bfloat16
I32 = jnp.int32

_U = np.zeros((D,), np.int32)
for _cc in range(D // 32):
    for _k in range(16):
        _U[32 * _cc + _k] = 32 * _cc + 2 * _k
        _U[32 * _cc + 16 + _k] = 32 * _cc + 2 * _k + 1
_UINV = np.argsort(_U).astype(np.int32)

_mesh = plsc.VectorSubcoreMesh(core_axis_name="c", subcore_axis_name="s")
_sc_params = pltpu.CompilerParams(needs_layout_passes=False)


def _rsqrt_newton(ss):
    i = plsc.bitcast(ss, I32)
    i = jnp.int32(0x5F3759DF) - lax.shift_right_arithmetic(i, 1)
    r = plsc.bitcast(i, F32)
    r = r * (1.5 - 0.5 * ss * r * r)
    r = r * (1.5 - 0.5 * ss * r * r)
    return r


def _edge_w16(posx_v, posy_v, s16, t16):
    dx = plsc.load_gather(posx_v, [s16]) - plsc.load_gather(posx_v, [t16])
    dy = plsc.load_gather(posy_v, [s16]) - plsc.load_gather(posy_v, [t16])
    ss = dx * dx + dy * dy
    dist = ss * _rsqrt_newton(ss)
    return 1.0 - dist * INV_SQRT2


@functools.partial(
    pl.kernel,
    out_type=(
        jax.ShapeDtypeStruct((E0P // 128, 128), F32),
        jax.ShapeDtypeStruct((E1P // 128, 128), F32),
        jax.ShapeDtypeStruct((2 * NP,), F32),
        jax.ShapeDtypeStruct((2 * NP,), F32),
    ),
    mesh=_mesh,
    compiler_params=_sc_params,
    scratch_types=[
        pltpu.VMEM((NP,), F32),
        pltpu.VMEM((NP,), F32),
        pltpu.VMEM((NC1, 128), I32),
        pltpu.VMEM((NC1, 128), I32),
        pltpu.VMEM((NC1, 128), F32),
        pltpu.VMEM((ROWS_PER_TILE,), F32),
        pltpu.VMEM_SHARED((NP,), F32),
        pltpu.VMEM_SHARED((NP,), F32),
    ],
)
def _k_edges(posx_h, posy_h, s0_h, t0_h, s1_h, t1_h,
             w0_h, w1_h, degp0_h, degp1_h,
             posx_v, posy_v, src_v, dst_v, w_v, z_v, deg0_sp, deg1_sp):
    c = lax.axis_index("c")
    s = lax.axis_index("s")
    wid = c * 16 + s
    pltpu.sync_copy(posx_h, posx_v)
    pltpu.sync_copy(posy_h, posy_v)
    zero16 = jnp.zeros((16,), F32)
    for i in range(ROWS_PER_TILE // 16):
        z_v[pl.ds(i * 16, 16)] = zero16
    pltpu.sync_copy(z_v, deg0_sp.at[pl.ds(s * ROWS_PER_TILE, ROWS_PER_TILE)])
    pltpu.sync_copy(z_v, deg1_sp.at[pl.ds(s * ROWS_PER_TILE, ROWS_PER_TILE)])
    plsc.subcore_barrier()

    def run_layer(s_h, t_h, w_h, deg_sp, nc):
        pltpu.sync_copy(s_h.at[pl.ds(wid * nc, nc)], src_v.at[pl.ds(0, nc)])
        pltpu.sync_copy(t_h.at[pl.ds(wid * nc, nc)], dst_v.at[pl.ds(0, nc)])

        def chunk(g, _):
            for j in range(8):
                s16 = src_v[g, pl.ds(j * 16, 16)]
                t16 = dst_v[g, pl.ds(j * 16, 16)]
                w_v[g, pl.ds(j * 16, 16)] = _edge_w16(posx_v, posy_v, s16, t16)
            pltpu.sync_copy(w_v.at[g], deg_sp.at[dst_v.at[g]], add=True)
            return _

        lax.fori_loop(0, nc, chunk, None)
        pltpu.sync_copy(w_v.at[pl.ds(0, nc)], w_h.at[pl.ds(wid * nc, nc)])

    run_layer(s0_h, t0_h, w0_h, deg0_sp, NC0)
    run_layer(s1_h, t1_h, w1_h, deg1_sp, NC1)
    plsc.subcore_barrier()
    off = s * ROWS_PER_TILE
    pltpu.sync_copy(deg0_sp.at[pl.ds(off, ROWS_PER_TILE)],
                    degp0_h.at[pl.ds(c * NP + off, ROWS_PER_TILE)])
    pltpu.sync_copy(deg1_sp.at[pl.ds(off, ROWS_PER_TILE)],
                    degp1_h.at[pl.ds(c * NP + off, ROWS_PER_TILE)])


def _make_aggregate(nc):
    assert nc % 2 == 0

    @functools.partial(
        pl.kernel,
        out_type=jax.ShapeDtypeStruct((2 * NP, D), F32),
        mesh=_mesh,
        compiler_params=_sc_params,
        scratch_types=[
            pltpu.VMEM((nc, 128), I32),
            pltpu.VMEM((2, 128), I32),
            pltpu.VMEM((2, 128), F32),
            pltpu.VMEM((128, D), F32),
            pltpu.VMEM((128, D), F32),
            pltpu.SemaphoreType.DMA,
            pltpu.SemaphoreType.DMA,
            pltpu.SemaphoreType.DMA,
            pltpu.SemaphoreType.DMA,
            pltpu.VMEM_SHARED((NP, D), F32),
        ],
    )
    def k(y_h, s_h, t_h, w_h, out_h, src_v, dst_v, w_v, rows_a, rows_b,
          gsem_a, gsem_b, isem_a, isem_b, acc_sp):
        c = lax.axis_index("c")
        s = lax.axis_index("s")
        wid = c * 16 + s
        base = wid * nc
        pltpu.sync_copy(s_h.at[pl.ds(base, nc)], src_v)

        zero16 = jnp.zeros((16,), F32)

        def zrow(j, _):
            for cc in range(D // 16):
                rows_a[j, pl.ds(cc * 16, 16)] = zero16
            return _

        lax.fori_loop(0, 128, zrow, None)
        for b in range(ROWS_PER_TILE // 128):
            pltpu.sync_copy(
                rows_a, acc_sp.at[pl.ds(s * ROWS_PER_TILE + b * 128, 128)])
        plsc.subcore_barrier()

        rows = (rows_a, rows_b)
        gsem = (gsem_a, gsem_b)
        isem = (isem_a, isem_b)

        def gather_issue(g, b):
            for q in range(4):
                pltpu.async_copy(y_h.at[src_v.at[g, pl.ds(q * 32, 32)]],
                                 rows[b].at[pl.ds(q * 32, 32)], gsem[b])

        def gather_wait(g, b):
            for q in range(4):
                pltpu.make_async_copy(
                    y_h.at[src_v.at[g, pl.ds(q * 32, 32)]],
                    rows[b].at[pl.ds(q * 32, 32)], gsem[b]).wait()

        def idx_issue(g, b):
            pltpu.async_copy(t_h.at[base + g], dst_v.at[b], isem[b])
            pltpu.async_copy(w_h.at[base + g], w_v.at[b], isem[b])

        def idx_wait(b):
            pltpu.make_async_copy(t_h.at[0], dst_v.at[b], isem[b]).wait()
            pltpu.make_async_copy(w_h.at[0], w_v.at[b], isem[b]).wait()

        for b in range(2):
            idx_issue(b, b)
            gather_issue(b, b)

        def scale(b):
            def scale_16rows(jj, _):
                j0 = jj * 16
                w16 = w_v[b, pl.ds(j0, 16)]
                for r in range(16):
                    wb = jnp.broadcast_to(w16[r], (16,))
                    for cc in range(D // 16):
                        sl = pl.ds(cc * 16, 16)
                        rows[b][j0 + r, sl] = rows[b][j0 + r, sl] * wb
                return _

            lax.fori_loop(0, 8, scale_16rows, None)

        def pair(i, _):
            for b in range(2):
                g = 2 * i + b
                gather_wait(g, b)
                idx_wait(b)
                scale(b)
                pltpu.sync_copy(rows[b], acc_sp.at[dst_v.at[b]], add=True)
                have_next = i < nc // 2 - 1

                @pl.when(have_next)
                def _next():
                    gather_issue(g + 2, b)
                    idx_issue(g + 2, b)

            return _

        lax.fori_loop(0, nc // 2, pair, None)
        plsc.subcore_barrier()
        off = s * ROWS_PER_TILE
        pltpu.sync_copy(acc_sp.at[pl.ds(off, ROWS_PER_TILE)],
                        out_h.at[pl.ds(c * NP + off, ROWS_PER_TILE)])

    return k


_agg0 = _make_aggregate(NC0)
_agg1 = _make_aggregate(NC1)


def _t2_body(xp, w0p, dpa0, dpb0, dpa1, dpb1, xw0, y0, d0, d1):
    xw = jnp.dot(xp[...], w0p[...], preferred_element_type=F32)
    xw0[...] = xw
    dv0 = lax.rsqrt(dpa0[...] + dpb0[...] + 1.0)
    d0[...] = dv0
    d1[...] = lax.rsqrt(dpa1[...] + dpb1[...] + 1.0)
    y0[...] = xw * dv0


def _t3_body(p0a, p0b, d0, xw0, b0r, g0r, be0r, w1p, d1, xw1, y1):
    dv = d0[...]
    h = dv * (p0a[...] + p0b[...]) + dv * dv * xw0[...] + b0r[...]
    rmask = lax.broadcasted_iota(I32, (NP, 1), 0) < N
    h = jnp.where(rmask, h, 0.0)
    mean = jnp.sum(h, axis=0, keepdims=True) * (1.0 / N)
    cent = h - mean
    var = jnp.sum(jnp.where(rmask, cent * cent, 0.0), axis=0,
                  keepdims=True) * (1.0 / N)
    hbn = cent * lax.rsqrt(var + 1e-5) * g0r[...] + be0r[...]
    hbn = jnp.where(rmask, hbn, 0.0)
    xwv = jnp.dot(hbn, w1p[...], preferred_element_type=F32)
    xw1[...] = xwv
    y1[...] = xwv * d1[...]


def _t4_body(p1a, p1b, d1, xw1, b1r, out):
    dv = d1[...]
    out[...] = dv * (p1a[...] + p1b[...]) + dv * dv * xw1[...] + b1r[...]


def _pad_edges(ei, ep):
    e = ei.shape[1]
    pad = 10000 + (jnp.arange(ep - e, dtype=I32) % 240)
    src = jnp.concatenate([ei[0].astype(I32), pad]).reshape(ep // 128, 128)
    dst = jnp.concatenate([ei[1].astype(I32), pad]).reshape(ep // 128, 128)
    return src, dst


def kernel(x, pos, edge_index0, edge_index1, W0, b0, gamma0, beta0, W1, b1):
    f = jnp.zeros
    xp = f((NP, D), F32).at[:N].set(x)
    posx = f((NP,), F32).at[:N].set(pos[:, 0])
    posy = f((NP,), F32).at[:N].set(pos[:, 1])
    w0p = f((D, D), F32).at[:, : W0.shape[1]].set(W0)
    w1p = f((D, D), F32).at[: W1.shape[0], :].set(W1)
    b0r = f((1, D), F32).at[0, : b0.shape[0]].set(b0)
    g0r = f((1, D), F32).at[0, : gamma0.shape[0]].set(gamma0)
    be0r = f((1, D), F32).at[0, : beta0.shape[0]].set(beta0)
    b1r = b1.reshape(1, D)
    s0, t0 = _pad_edges(edge_index0, E0P)
    s1, t1 = _pad_edges(edge_index1, E1P)

    ew0, ew1, degp0, degp1 = _k_edges(posx, posy, s0, t0, s1, t1)

    sds = jax.ShapeDtypeStruct
    xw0, y0, d0, d1 = pl.pallas_call(
        _t2_body,
        out_shape=(sds((NP, D), F32), sds((NP, D), F32),
                   sds((NP, 1), F32), sds((NP, 1), F32)),
    )(xp, w0p,
      degp0[:NP].reshape(NP, 1), degp0[NP:].reshape(NP, 1),
      degp1[:NP].reshape(NP, 1), degp1[NP:].reshape(NP, 1))

    p0 = _agg0(y0, s0, t0, ew0)
    xw1, y1 = pl.pallas_call(
        _t3_body,
        out_shape=(sds((NP, D), F32), sds((NP, D), F32)),
    )(p0[:NP], p0[NP:], d0, xw0, b0r, g0r, be0r, w1p, d1)

    p1 = _agg1(y1, s1, t1, ew1)
    out = pl.pallas_call(
        _t4_body,
        out_shape=sds((NP, D), F32),
    )(p1[:NP], p1[NP:], d1, xw1, b1r)
    return out[:N]


# speedup vs baseline: 1.0275x; 1.0275x over previous
import functools

import jax
import jax.numpy as jnp
import numpy as np
from jax import lax
from jax.experimental import pallas as pl
from jax.experimental.pallas import tpu as pltpu
from jax.experimental.pallas import tpu_sc as plsc

N = 10000
NP = 10240
D = 128
E0P = 32768
E1P = 327680
NC0 = E0P // (32 * 128)
NC1 = E1P // (32 * 128)
ROWS_PER_TILE = NP // 16
INV_SQRT2 = np.float32(1.0 / np.sqrt(2.0))
F32 = jnp.float32
BF16 = jnp.bfloat16
I32 = jnp.int32

_U = np.zeros((D,), np.int32)
for _cc in range(D // 32):
    for _k in range(16):
        _U[32 * _cc + _k] = 32 * _cc + 2 * _k
        _U[32 * _cc + 16 + _k] = 32 * _cc + 2 * _k + 1
_UINV = np.argsort(_U).astype(np.int32)

_mesh = plsc.VectorSubcoreMesh(core_axis_name="c", subcore_axis_name="s")
_sc_params = pltpu.CompilerParams(needs_layout_passes=False)


def _rsqrt_newton(ss):
    i = plsc.bitcast(ss, I32)
    i = jnp.int32(0x5F3759DF) - lax.shift_right_arithmetic(i, 1)
    r = plsc.bitcast(i, F32)
    r = r * (1.5 - 0.5 * ss * r * r)
    r = r * (1.5 - 0.5 * ss * r * r)
    return r


def _edge_w16(posx_v, posy_v, s16, t16):
    dx = plsc.load_gather(posx_v, [s16]) - plsc.load_gather(posx_v, [t16])
    dy = plsc.load_gather(posy_v, [s16]) - plsc.load_gather(posy_v, [t16])
    ss = dx * dx + dy * dy
    dist = ss * _rsqrt_newton(ss)
    return 1.0 - dist * INV_SQRT2


@functools.partial(
    pl.kernel,
    out_type=(
        jax.ShapeDtypeStruct((E0P // 128, 128), F32),
        jax.ShapeDtypeStruct((E1P // 128, 128), F32),
        jax.ShapeDtypeStruct((2 * NP,), F32),
        jax.ShapeDtypeStruct((2 * NP,), F32),
    ),
    mesh=_mesh,
    compiler_params=_sc_params,
    scratch_types=[
        pltpu.VMEM((NP,), F32),
        pltpu.VMEM((NP,), F32),
        pltpu.VMEM((NC1, 128), I32),
        pltpu.VMEM((NC1, 128), I32),
        pltpu.VMEM((NC1, 128), F32),
        pltpu.VMEM((ROWS_PER_TILE,), F32),
        pltpu.VMEM_SHARED((NP,), F32),
        pltpu.VMEM_SHARED((NP,), F32),
        pltpu.SemaphoreType.DMA,
    ],
)
def _k_edges(posx_h, posy_h, s0_h, t0_h, s1_h, t1_h,
             w0_h, w1_h, degp0_h, degp1_h,
             posx_v, posy_v, src_v, dst_v, w_v, z_v, deg0_sp, deg1_sp, dsem):
    c = lax.axis_index("c")
    s = lax.axis_index("s")
    wid = c * 16 + s
    pltpu.sync_copy(posx_h, posx_v)
    pltpu.sync_copy(posy_h, posy_v)
    zero16 = jnp.zeros((16,), F32)
    for i in range(ROWS_PER_TILE // 16):
        z_v[pl.ds(i * 16, 16)] = zero16
    pltpu.sync_copy(z_v, deg0_sp.at[pl.ds(s * ROWS_PER_TILE, ROWS_PER_TILE)])
    pltpu.sync_copy(z_v, deg1_sp.at[pl.ds(s * ROWS_PER_TILE, ROWS_PER_TILE)])
    plsc.subcore_barrier()

    def run_layer(s_h, t_h, w_h, deg_sp, nc):
        pltpu.sync_copy(s_h.at[pl.ds(wid * nc, nc)], src_v.at[pl.ds(0, nc)])
        pltpu.sync_copy(t_h.at[pl.ds(wid * nc, nc)], dst_v.at[pl.ds(0, nc)])

        def chunk(g, _):
            for j in range(8):
                s16 = src_v[g, pl.ds(j * 16, 16)]
                t16 = dst_v[g, pl.ds(j * 16, 16)]
                w_v[g, pl.ds(j * 16, 16)] = _edge_w16(posx_v, posy_v, s16, t16)
            pltpu.async_copy(w_v.at[g], deg_sp.at[dst_v.at[g]], dsem,
                             add=True)
            return _

        lax.fori_loop(0, nc, chunk, None)
        pltpu.sync_copy(w_v.at[pl.ds(0, nc)], w_h.at[pl.ds(wid * nc, nc)])

        def drain(g, _):
            pltpu.make_async_copy(w_v.at[g], deg_sp.at[dst_v.at[g]],
                                  dsem).wait()
            return _

        lax.fori_loop(0, nc, drain, None)

    run_layer(s0_h, t0_h, w0_h, deg0_sp, NC0)
    run_layer(s1_h, t1_h, w1_h, deg1_sp, NC1)
    plsc.subcore_barrier()
    off = s * ROWS_PER_TILE
    pltpu.sync_copy(deg0_sp.at[pl.ds(off, ROWS_PER_TILE)],
                    degp0_h.at[pl.ds(c * NP + off, ROWS_PER_TILE)])
    pltpu.sync_copy(deg1_sp.at[pl.ds(off, ROWS_PER_TILE)],
                    degp1_h.at[pl.ds(c * NP + off, ROWS_PER_TILE)])


def _make_aggregate(nc):
    assert nc % 2 == 0

    @functools.partial(
        pl.kernel,
        out_type=jax.ShapeDtypeStruct((2 * NP, D), F32),
        mesh=_mesh,
        compiler_params=_sc_params,
        scratch_types=[
            pltpu.VMEM((nc, 128), I32),
            pltpu.VMEM((2, 128), I32),
            pltpu.VMEM((2, 128), F32),
            pltpu.VMEM((128, D), F32),
            pltpu.VMEM((128, D), F32),
            pltpu.SemaphoreType.DMA,
            pltpu.SemaphoreType.DMA,
            pltpu.SemaphoreType.DMA,
            pltpu.SemaphoreType.DMA,
            pltpu.VMEM_SHARED((NP, D), F32),
        ],
    )
    def k(y_h, s_h, t_h, w_h, out_h, src_v, dst_v, w_v, rows_a, rows_b,
          gsem_a, gsem_b, isem_a, isem_b, acc_sp):
        c = lax.axis_index("c")
        s = lax.axis_index("s")
        wid = c * 16 + s
        base = wid * nc
        pltpu.sync_copy(s_h.at[pl.ds(base, nc)], src_v)

        zero16 = jnp.zeros((16,), F32)

        def zrow(j, _):
            for cc in range(D // 16):
                rows_a[j, pl.ds(cc * 16, 16)] = zero16
            return _

        lax.fori_loop(0, 128, zrow, None)
        for b in range(ROWS_PER_TILE // 128):
            pltpu.sync_copy(
                rows_a, acc_sp.at[pl.ds(s * ROWS_PER_TILE + b * 128, 128)])
        plsc.subcore_barrier()

        rows = (rows_a, rows_b)
        gsem = (gsem_a, gsem_b)
        isem = (isem_a, isem_b)

        def gather_issue(g, b):
            for q in range(2):
                pltpu.async_copy(y_h.at[src_v.at[g, pl.ds(q * 64, 64)]],
                                 rows[b].at[pl.ds(q * 64, 64)], gsem[b])

        def gather_wait(g, b):
            for q in range(2):
                pltpu.make_async_copy(
                    y_h.at[src_v.at[g, pl.ds(q * 64, 64)]],
                    rows[b].at[pl.ds(q * 64, 64)], gsem[b]).wait()

        def idx_issue(g, b):
            pltpu.async_copy(t_h.at[base + g], dst_v.at[b], isem[b])
            pltpu.async_copy(w_h.at[base + g], w_v.at[b], isem[b])

        def idx_wait(b):
            pltpu.make_async_copy(t_h.at[0], dst_v.at[b], isem[b]).wait()
            pltpu.make_async_copy(w_h.at[0], w_v.at[b], isem[b]).wait()

        for b in range(2):
            idx_issue(b, b)
            gather_issue(b, b)

        def scale(b):
            def scale_16rows(jj, _):
                j0 = jj * 16
                w16 = w_v[b, pl.ds(j0, 16)]
                for r in range(16):
                    wb = jnp.broadcast_to(w16[r], (16,))
                    for cc in range(D // 16):
                        sl = pl.ds(cc * 16, 16)
                        rows[b][j0 + r, sl] = rows[b][j0 + r, sl] * wb
                return _

            lax.fori_loop(0, 8, scale_16rows, None)

        def pair(i, _):
            for b in range(2):
                g = 2 * i + b
                gather_wait(g, b)
                idx_wait(b)
                scale(b)
                pltpu.sync_copy(rows[b], acc_sp.at[dst_v.at[b]], add=True)
                have_next = i < nc // 2 - 1

                @pl.when(have_next)
                def _next():
                    gather_issue(g + 2, b)
                    idx_issue(g + 2, b)

            return _

        lax.fori_loop(0, nc // 2, pair, None)
        plsc.subcore_barrier()
        off = s * ROWS_PER_TILE
        pltpu.sync_copy(acc_sp.at[pl.ds(off, ROWS_PER_TILE)],
                        out_h.at[pl.ds(c * NP + off, ROWS_PER_TILE)])

    return k


_agg0 = _make_aggregate(NC0)
_agg1 = _make_aggregate(NC1)


def _t2_body(xp, w0p, dpa0, dpb0, dpa1, dpb1, xw0, y0, d0, d1):
    xw = jnp.dot(xp[...], w0p[...], preferred_element_type=F32)
    xw0[...] = xw
    dv0 = lax.rsqrt(dpa0[...] + dpb0[...] + 1.0)
    d0[...] = dv0
    d1[...] = lax.rsqrt(dpa1[...] + dpb1[...] + 1.0)
    y0[...] = xw * dv0


def _t3_body(p0a, p0b, d0, xw0, b0r, g0r, be0r, w1p, d1, xw1, y1):
    dv = d0[...]
    h = dv * (p0a[...] + p0b[...]) + dv * dv * xw0[...] + b0r[...]
    rmask = lax.broadcasted_iota(I32, (NP, 1), 0) < N
    h = jnp.where(rmask, h, 0.0)
    mean = jnp.sum(h, axis=0, keepdims=True) * (1.0 / N)
    cent = h - mean
    var = jnp.sum(jnp.where(rmask, cent * cent, 0.0), axis=0,
                  keepdims=True) * (1.0 / N)
    hbn = cent * lax.rsqrt(var + 1e-5) * g0r[...] + be0r[...]
    hbn = jnp.where(rmask, hbn, 0.0)
    xwv = jnp.dot(hbn, w1p[...], preferred_element_type=F32)
    xw1[...] = xwv
    y1[...] = xwv * d1[...]


def _t4_body(p1a, p1b, d1, xw1, b1r, out):
    dv = d1[...]
    out[...] = dv * (p1a[...] + p1b[...]) + dv * dv * xw1[...] + b1r[...]


def _pad_edges(ei, ep):
    e = ei.shape[1]
    pad = 10000 + (jnp.arange(ep - e, dtype=I32) % 240)
    src = jnp.concatenate([ei[0].astype(I32), pad]).reshape(ep // 128, 128)
    dst = jnp.concatenate([ei[1].astype(I32), pad]).reshape(ep // 128, 128)
    return src, dst


def kernel(x, pos, edge_index0, edge_index1, W0, b0, gamma0, beta0, W1, b1):
    f = jnp.zeros
    xp = f((NP, D), F32).at[:N].set(x)
    posx = f((NP,), F32).at[:N].set(pos[:, 0])
    posy = f((NP,), F32).at[:N].set(pos[:, 1])
    w0p = f((D, D), F32).at[:, : W0.shape[1]].set(W0)
    w1p = f((D, D), F32).at[: W1.shape[0], :].set(W1)
    b0r = f((1, D), F32).at[0, : b0.shape[0]].set(b0)
    g0r = f((1, D), F32).at[0, : gamma0.shape[0]].set(gamma0)
    be0r = f((1, D), F32).at[0, : beta0.shape[0]].set(beta0)
    b1r = b1.reshape(1, D)
    s0, t0 = _pad_edges(edge_index0, E0P)
    s1, t1 = _pad_edges(edge_index1, E1P)

    ew0, ew1, degp0, degp1 = _k_edges(posx, posy, s0, t0, s1, t1)

    sds = jax.ShapeDtypeStruct
    xw0, y0, d0, d1 = pl.pallas_call(
        _t2_body,
        out_shape=(sds((NP, D), F32), sds((NP, D), F32),
                   sds((NP, 1), F32), sds((NP, 1), F32)),
    )(xp, w0p,
      degp0[:NP].reshape(NP, 1), degp0[NP:].reshape(NP, 1),
      degp1[:NP].reshape(NP, 1), degp1[NP:].reshape(NP, 1))

    p0 = _agg0(y0, s0, t0, ew0)
    xw1, y1 = pl.pallas_call(
        _t3_body,
        out_shape=(sds((NP, D), F32), sds((NP, D), F32)),
    )(p0[:NP], p0[NP:], d0, xw0, b0r, g0r, be0r, w1p, d1)

    p1 = _agg1(y1, s1, t1, ew1)
    out = pl.pallas_call(
        _t4_body,
        out_shape=sds((NP, D), F32),
    )(p1[:NP], p1[NP:], d1, xw1, b1r)
    return out[:N]


# speedup vs baseline: 1.0369x; 1.0091x over previous
import functools

import jax
import jax.numpy as jnp
import numpy as np
from jax import lax
from jax.experimental import pallas as pl
from jax.experimental.pallas import tpu as pltpu
from jax.experimental.pallas import tpu_sc as plsc

N = 10000
NP = 10240
D = 128
E0P = 32768
E1P = 327680
NC0 = E0P // (32 * 128)
NC1 = E1P // (32 * 128)
ROWS_PER_TILE = NP // 16
INV_SQRT2 = np.float32(1.0 / np.sqrt(2.0))
F32 = jnp.float32
BF16 = jnp.bfloat16
I32 = jnp.int32

_U = np.zeros((D,), np.int32)
for _cc in range(D // 32):
    for _k in range(16):
        _U[32 * _cc + _k] = 32 * _cc + 2 * _k
        _U[32 * _cc + 16 + _k] = 32 * _cc + 2 * _k + 1
_UINV = np.argsort(_U).astype(np.int32)

_mesh = plsc.VectorSubcoreMesh(core_axis_name="c", subcore_axis_name="s")
_sc_params = pltpu.CompilerParams(needs_layout_passes=False)


def _rsqrt_newton(ss):
    i = plsc.bitcast(ss, I32)
    i = jnp.int32(0x5F3759DF) - lax.shift_right_arithmetic(i, 1)
    r = plsc.bitcast(i, F32)
    r = r * (1.5 - 0.5 * ss * r * r)
    r = r * (1.5 - 0.5 * ss * r * r)
    return r


def _edge_w16(posx_v, posy_v, s16, t16):
    dx = plsc.load_gather(posx_v, [s16]) - plsc.load_gather(posx_v, [t16])
    dy = plsc.load_gather(posy_v, [s16]) - plsc.load_gather(posy_v, [t16])
    ss = dx * dx + dy * dy
    dist = ss * _rsqrt_newton(ss)
    return 1.0 - dist * INV_SQRT2


@functools.partial(
    pl.kernel,
    out_type=(
        jax.ShapeDtypeStruct((E0P // 128, 128), F32),
        jax.ShapeDtypeStruct((E1P // 128, 128), F32),
        jax.ShapeDtypeStruct((2 * NP,), F32),
        jax.ShapeDtypeStruct((2 * NP,), F32),
    ),
    mesh=_mesh,
    compiler_params=_sc_params,
    scratch_types=[
        pltpu.VMEM((NP,), F32),
        pltpu.VMEM((NP,), F32),
        pltpu.VMEM((NC1, 128), I32),
        pltpu.VMEM((NC1, 128), I32),
        pltpu.VMEM((NC1, 128), F32),
        pltpu.VMEM((ROWS_PER_TILE,), F32),
        pltpu.VMEM_SHARED((NP,), F32),
        pltpu.VMEM_SHARED((NP,), F32),
        pltpu.SemaphoreType.DMA,
    ],
)
def _k_edges(posx_h, posy_h, s0_h, t0_h, s1_h, t1_h,
             w0_h, w1_h, degp0_h, degp1_h,
             posx_v, posy_v, src_v, dst_v, w_v, z_v, deg0_sp, deg1_sp, dsem):
    c = lax.axis_index("c")
    s = lax.axis_index("s")
    wid = c * 16 + s
    pltpu.async_copy(posx_h, posx_v, dsem)
    pltpu.async_copy(posy_h, posy_v, dsem)
    zero16 = jnp.zeros((16,), F32)
    for i in range(ROWS_PER_TILE // 16):
        z_v[pl.ds(i * 16, 16)] = zero16
    pltpu.sync_copy(z_v, deg0_sp.at[pl.ds(s * ROWS_PER_TILE, ROWS_PER_TILE)])
    pltpu.sync_copy(z_v, deg1_sp.at[pl.ds(s * ROWS_PER_TILE, ROWS_PER_TILE)])
    pltpu.make_async_copy(posx_h, posx_v, dsem).wait()
    pltpu.make_async_copy(posy_h, posy_v, dsem).wait()
    plsc.subcore_barrier()

    def run_layer(s_h, t_h, w_h, deg_sp, nc):
        pltpu.sync_copy(s_h.at[pl.ds(wid * nc, nc)], src_v.at[pl.ds(0, nc)])
        pltpu.sync_copy(t_h.at[pl.ds(wid * nc, nc)], dst_v.at[pl.ds(0, nc)])

        def chunk(g, _):
            for j in range(8):
                s16 = src_v[g, pl.ds(j * 16, 16)]
                t16 = dst_v[g, pl.ds(j * 16, 16)]
                w_v[g, pl.ds(j * 16, 16)] = _edge_w16(posx_v, posy_v, s16, t16)
            pltpu.async_copy(w_v.at[g], deg_sp.at[dst_v.at[g]], dsem,
                             add=True)
            return _

        lax.fori_loop(0, nc, chunk, None)
        pltpu.sync_copy(w_v.at[pl.ds(0, nc)], w_h.at[pl.ds(wid * nc, nc)])

        def drain(g, _):
            pltpu.make_async_copy(w_v.at[g], deg_sp.at[dst_v.at[g]],
                                  dsem).wait()
            return _

        lax.fori_loop(0, nc, drain, None)

    run_layer(s0_h, t0_h, w0_h, deg0_sp, NC0)
    run_layer(s1_h, t1_h, w1_h, deg1_sp, NC1)
    plsc.subcore_barrier()
    off = s * ROWS_PER_TILE
    pltpu.sync_copy(deg0_sp.at[pl.ds(off, ROWS_PER_TILE)],
                    degp0_h.at[pl.ds(c * NP + off, ROWS_PER_TILE)])
    pltpu.sync_copy(deg1_sp.at[pl.ds(off, ROWS_PER_TILE)],
                    degp1_h.at[pl.ds(c * NP + off, ROWS_PER_TILE)])


def _make_aggregate(nc):
    assert nc % 2 == 0

    @functools.partial(
        pl.kernel,
        out_type=jax.ShapeDtypeStruct((2 * NP, D), F32),
        mesh=_mesh,
        compiler_params=_sc_params,
        scratch_types=[
            pltpu.VMEM((nc, 128), I32),
            pltpu.VMEM((2, 128), I32),
            pltpu.VMEM((2, 128), F32),
            pltpu.VMEM((128, D), F32),
            pltpu.VMEM((128, D), F32),
            pltpu.SemaphoreType.DMA,
            pltpu.SemaphoreType.DMA,
            pltpu.SemaphoreType.DMA,
            pltpu.SemaphoreType.DMA,
            pltpu.VMEM_SHARED((NP, D), F32),
        ],
    )
    def k(y_h, s_h, t_h, w_h, out_h, src_v, dst_v, w_v, rows_a, rows_b,
          gsem_a, gsem_b, isem_a, isem_b, acc_sp):
        c = lax.axis_index("c")
        s = lax.axis_index("s")
        wid = c * 16 + s
        base = wid * nc
        pltpu.sync_copy(s_h.at[pl.ds(base, nc)], src_v)

        zero16 = jnp.zeros((16,), F32)

        def zrow(j, _):
            for cc in range(D // 16):
                rows_a[j, pl.ds(cc * 16, 16)] = zero16
            return _

        lax.fori_loop(0, 128, zrow, None)
        for b in range(ROWS_PER_TILE // 128):
            pltpu.async_copy(
                rows_a, acc_sp.at[pl.ds(s * ROWS_PER_TILE + b * 128, 128)],
                gsem_a)
        for b in range(ROWS_PER_TILE // 128):
            pltpu.make_async_copy(
                rows_a, acc_sp.at[pl.ds(s * ROWS_PER_TILE + b * 128, 128)],
                gsem_a).wait()
        plsc.subcore_barrier()

        rows = (rows_a, rows_b)
        gsem = (gsem_a, gsem_b)
        isem = (isem_a, isem_b)

        def gather_issue(g, b):
            for q in range(2):
                pltpu.async_copy(y_h.at[src_v.at[g, pl.ds(q * 64, 64)]],
                                 rows[b].at[pl.ds(q * 64, 64)], gsem[b])

        def gather_wait(g, b):
            for q in range(2):
                pltpu.make_async_copy(
                    y_h.at[src_v.at[g, pl.ds(q * 64, 64)]],
                    rows[b].at[pl.ds(q * 64, 64)], gsem[b]).wait()

        def idx_issue(g, b):
            pltpu.async_copy(t_h.at[base + g], dst_v.at[b], isem[b])
            pltpu.async_copy(w_h.at[base + g], w_v.at[b], isem[b])

        def idx_wait(b):
            pltpu.make_async_copy(t_h.at[0], dst_v.at[b], isem[b]).wait()
            pltpu.make_async_copy(w_h.at[0], w_v.at[b], isem[b]).wait()

        for b in range(2):
            idx_issue(b, b)
            gather_issue(b, b)

        def scale(b):
            def scale_16rows(jj, _):
                j0 = jj * 16
                w16 = w_v[b, pl.ds(j0, 16)]
                for r in range(16):
                    wb = jnp.broadcast_to(w16[r], (16,))
                    for cc in range(D // 16):
                        sl = pl.ds(cc * 16, 16)
                        rows[b][j0 + r, sl] = rows[b][j0 + r, sl] * wb
                return _

            lax.fori_loop(0, 8, scale_16rows, None)

        def pair(i, _):
            for b in range(2):
                g = 2 * i + b
                gather_wait(g, b)
                idx_wait(b)
                scale(b)
                pltpu.sync_copy(rows[b], acc_sp.at[dst_v.at[b]], add=True)
                have_next = i < nc // 2 - 1

                @pl.when(have_next)
                def _next():
                    gather_issue(g + 2, b)
                    idx_issue(g + 2, b)

            return _

        lax.fori_loop(0, nc // 2, pair, None)
        plsc.subcore_barrier()
        off = s * ROWS_PER_TILE
        pltpu.sync_copy(acc_sp.at[pl.ds(off, ROWS_PER_TILE)],
                        out_h.at[pl.ds(c * NP + off, ROWS_PER_TILE)])

    return k


_agg0 = _make_aggregate(NC0)
_agg1 = _make_aggregate(NC1)


def _t2_body(xp, w0p, dpa0, dpb0, dpa1, dpb1, xw0, y0, d0, d1):
    xw = jnp.dot(xp[...], w0p[...], preferred_element_type=F32)
    xw0[...] = xw
    dv0 = lax.rsqrt(dpa0[...] + dpb0[...] + 1.0)
    d0[...] = dv0
    d1[...] = lax.rsqrt(dpa1[...] + dpb1[...] + 1.0)
    y0[...] = xw * dv0


def _t3_body(p0a, p0b, d0, xw0, b0r, g0r, be0r, w1p, d1, xw1, y1):
    dv = d0[...]
    h = dv * (p0a[...] + p0b[...]) + dv * dv * xw0[...] + b0r[...]
    rmask = lax.broadcasted_iota(I32, (NP, 1), 0) < N
    h = jnp.where(rmask, h, 0.0)
    mean = jnp.sum(h, axis=0, keepdims=True) * (1.0 / N)
    cent = h - mean
    var = jnp.sum(jnp.where(rmask, cent * cent, 0.0), axis=0,
                  keepdims=True) * (1.0 / N)
    hbn = cent * lax.rsqrt(var + 1e-5) * g0r[...] + be0r[...]
    hbn = jnp.where(rmask, hbn, 0.0)
    xwv = jnp.dot(hbn, w1p[...], preferred_element_type=F32)
    xw1[...] = xwv
    y1[...] = xwv * d1[...]


def _t4_body(p1a, p1b, d1, xw1, b1r, out):
    dv = d1[...]
    out[...] = dv * (p1a[...] + p1b[...]) + dv * dv * xw1[...] + b1r[...]


def _pad_edges(ei, ep):
    e = ei.shape[1]
    pad = 10000 + (jnp.arange(ep - e, dtype=I32) % 240)
    src = jnp.concatenate([ei[0].astype(I32), pad]).reshape(ep // 128, 128)
    dst = jnp.concatenate([ei[1].astype(I32), pad]).reshape(ep // 128, 128)
    return src, dst


def kernel(x, pos, edge_index0, edge_index1, W0, b0, gamma0, beta0, W1, b1):
    f = jnp.zeros
    xp = f((NP, D), F32).at[:N].set(x)
    posx = f((NP,), F32).at[:N].set(pos[:, 0])
    posy = f((NP,), F32).at[:N].set(pos[:, 1])
    w0p = f((D, D), F32).at[:, : W0.shape[1]].set(W0)
    w1p = f((D, D), F32).at[: W1.shape[0], :].set(W1)
    b0r = f((1, D), F32).at[0, : b0.shape[0]].set(b0)
    g0r = f((1, D), F32).at[0, : gamma0.shape[0]].set(gamma0)
    be0r = f((1, D), F32).at[0, : beta0.shape[0]].set(beta0)
    b1r = b1.reshape(1, D)
    s0, t0 = _pad_edges(edge_index0, E0P)
    s1, t1 = _pad_edges(edge_index1, E1P)

    ew0, ew1, degp0, degp1 = _k_edges(posx, posy, s0, t0, s1, t1)

    sds = jax.ShapeDtypeStruct
    xw0, y0, d0, d1 = pl.pallas_call(
        _t2_body,
        out_shape=(sds((NP, D), F32), sds((NP, D), F32),
                   sds((NP, 1), F32), sds((NP, 1), F32)),
    )(xp, w0p,
      degp0[:NP].reshape(NP, 1), degp0[NP:].reshape(NP, 1),
      degp1[:NP].reshape(NP, 1), degp1[NP:].reshape(NP, 1))

    p0 = _agg0(y0, s0, t0, ew0)
    xw1, y1 = pl.pallas_call(
        _t3_body,
        out_shape=(sds((NP, D), F32), sds((NP, D), F32)),
    )(p0[:NP], p0[NP:], d0, xw0, b0r, g0r, be0r, w1p, d1)

    p1 = _agg1(y1, s1, t1, ew1)
    out = pl.pallas_call(
        _t4_body,
        out_shape=sds((NP, D), F32),
    )(p1[:NP], p1[NP:], d1, xw1, b1r)
    return out[:N]
